# baseline (speedup 1.0000x reference)
import functools

import jax
import jax.numpy as jnp
from jax import lax
from jax.experimental import pallas as pl
from jax.experimental.pallas import tpu as pltpu
from jax.experimental.pallas import tpu_sc as plsc

N = 10000
E = 320000
IN_DIM = 64
H = 128
C = 32
L = 4

NC = 2
NS = 16
NW = NC * NS
CHUNK = 96
NCHUNK = 107
EPAD = NW * NCHUNK * CHUNK
EDGES_PER_TILE = EPAD // NW
NPAD = 10112
ROWS_PER_TILE = NPAD // NS
NSLOT = 3


def _segment_sum_sc(cur, packed3d, zeros):
    mesh = plsc.VectorSubcoreMesh(core_axis_name="c", subcore_axis_name="s")

    @functools.partial(
        pl.kernel,
        out_type=jax.ShapeDtypeStruct((NC, NPAD, H), jnp.float32),
        mesh=mesh,
        scratch_types=[
            pltpu.VMEM((1, NCHUNK * CHUNK), jnp.int32),
            pltpu.VMEM((NSLOT, CHUNK), jnp.int32),
            pltpu.VMEM((NSLOT, CHUNK), jnp.int32),
            pltpu.VMEM((NSLOT, CHUNK, H), jnp.float32),
            pltpu.VMEM_SHARED((NPAD, H), jnp.float32),
            [pltpu.SemaphoreType.DMA] * NSLOT,
            [pltpu.SemaphoreType.DMA] * NSLOT,
        ],
    )
    def seg_sum(cur_hbm, pk_hbm, zeros_hbm, out_hbm,
                pk_v, src_v, dst_v, rows_v, acc_sh, gsem, ssem):
        cid = lax.axis_index("c")
        sid = lax.axis_index("s")
        wid = sid * NC + cid
        row0 = sid * ROWS_PER_TILE
        pltpu.sync_copy(zeros_hbm.at[pl.ds(row0, ROWS_PER_TILE)],
                        acc_sh.at[pl.ds(row0, ROWS_PER_TILE)])
        pltpu.sync_copy(pk_hbm.at[wid], pk_v)
        plsc.subcore_barrier()

        def unpack_src(j, b):
            for k in range(CHUNK // 16):
                sl = pl.ds(j * CHUNK + k * 16, 16)
                src_v[b, pl.ds(k * 16, 16)] = lax.shift_right_logical(
                    pk_v[0, sl], 14)

        def unpack_dst(j, b):
            for k in range(CHUNK // 16):
                sl = pl.ds(j * CHUNK + k * 16, 16)
                dst_v[b, pl.ds(k * 16, 16)] = lax.bitwise_and(
                    pk_v[0, sl], 0x3FFF)

        def start_gather(j, b):
            unpack_src(j, b)
            pltpu.async_copy(cur_hbm.at[src_v.at[b]], rows_v.at[b], gsem[b])

        def wait_gather(b):
            pltpu.make_async_copy(cur_hbm.at[src_v.at[b]],
                                  rows_v.at[b], gsem[b]).wait()

        def start_scatter(j, b):
            unpack_dst(j, b)
            pltpu.async_copy(rows_v.at[b], acc_sh.at[dst_v.at[b]], ssem[b],
                             add=True)

        def wait_scatter(b):
            pltpu.make_async_copy(rows_v.at[b], acc_sh.at[dst_v.at[b]],
                                  ssem[b]).wait()

        start_gather(0, 0)
        start_gather(1, 1)

        def body(jj, carry):
            for u in range(NSLOT):
                j = NSLOT * jj + u
                wait_gather(u)
                start_scatter(j, u)
                bn = (u + 2) % NSLOT

                @pl.when(j > 0)
                def _():
                    wait_scatter(bn)

                start_gather(j + 2, bn)
            return carry

        lax.fori_loop(0, (NCHUNK - 2) // NSLOT, body, 0)
        wait_gather(0)
        start_scatter(NCHUNK - 2, 0)
        wait_gather(1)
        start_scatter(NCHUNK - 1, 1)
        wait_scatter(2)
        wait_scatter(0)
        wait_scatter(1)
        plsc.subcore_barrier()
        pltpu.sync_copy(acc_sh.at[pl.ds(row0, ROWS_PER_TILE)],
                        out_hbm.at[cid].at[pl.ds(row0, ROWS_PER_TILE)])

    return seg_sum(cur, packed3d, zeros)


def _bn(x, gamma, beta):
    mu = jnp.mean(x, axis=0, keepdims=True)
    var = jnp.mean((x - mu) ** 2, axis=0, keepdims=True)
    return gamma * (x - mu) * lax.rsqrt(var + 1e-5) + beta


def _init_tc(h2d, emb, predW0, predb):
    def body(h_ref, emb_ref, pw_ref, pb_ref, x_ref, s_ref):
        onehot = (h_ref[...] == lax.broadcasted_iota(jnp.int32, (N, IN_DIM), 1)
                  ).astype(jnp.float32)
        x = jnp.dot(onehot, emb_ref[...], preferred_element_type=jnp.float32,
                    precision=lax.Precision.HIGHEST)
        x_ref[...] = x
        s_ref[...] = (jnp.dot(x, pw_ref[...], preferred_element_type=jnp.float32)
                      + jnp.sum(pb_ref[...], axis=0, keepdims=True))

    return pl.pallas_call(
        body,
        out_shape=[jax.ShapeDtypeStruct((N, H), jnp.float32),
                   jax.ShapeDtypeStruct((N, C), jnp.float32)],
    )(h2d, emb, predW0, predb)


def _layer_tc(cur, part, score, eps_i, w1, b1, g1, be1, w2, b2, ga, ba, gl, bl, pw):
    def body(cur_ref, p_ref, s_ref, eps_ref, w1_ref, b1_ref, g1_ref, be1_ref,
             w2_ref, b2_ref, ga_ref, ba_ref, gl_ref, bl_ref, pw_ref,
             out_ref, sout_ref):
        cur_ = cur_ref[...]
        p0 = p_ref[0, pl.ds(0, N), :]
        p1 = p_ref[1, pl.ds(0, N), :]
        z = (1.0 + eps_ref[0, 0]) * cur_ + p0 + p1
        z = jnp.dot(z, w1_ref[...], preferred_element_type=jnp.float32) + b1_ref[...]
        z = jax.nn.relu(_bn(z, g1_ref[...], be1_ref[...]))
        z = jnp.dot(z, w2_ref[...], preferred_element_type=jnp.float32) + b2_ref[...]
        z = jax.nn.relu(_bn(z, ga_ref[...], ba_ref[...]))
        z = jax.nn.relu(_bn(z, gl_ref[...], bl_ref[...]))
        new = cur_ + z
        out_ref[...] = new
        sout_ref[...] = s_ref[...] + jnp.dot(new, pw_ref[...],
                                             preferred_element_type=jnp.float32)

    return pl.pallas_call(
        body,
        out_shape=[jax.ShapeDtypeStruct((N, H), jnp.float32),
                   jax.ShapeDtypeStruct((N, C), jnp.float32)],
    )(cur, part, score, eps_i, w1, b1, g1, be1, w2, b2, ga, ba, gl, bl, pw)


def kernel(h, edge_index, e, emb, eps, W1, b1, g1, be1, W2, b2, ga, ba, gl, bl,
           predW, predb):
    del e
    h2d = h.astype(jnp.int32).reshape(N, 1)
    npad_e = EPAD - E
    pad_ar = jnp.arange(npad_e, dtype=jnp.int32)
    pad_src = pad_ar % N
    pad_dst = N + pad_ar % (NPAD - N)
    src_all = jnp.concatenate([edge_index[0].astype(jnp.int32), pad_src])
    dst_all = jnp.concatenate([edge_index[1].astype(jnp.int32), pad_dst])
    packed3d = (src_all * 16384 + dst_all).reshape(NW, 1, NCHUNK * CHUNK)
    zeros = jnp.zeros((NPAD, H), jnp.float32)

    cur, score = _init_tc(h2d, emb, predW[0], predb)
    for i in range(L):
        part = _segment_sum_sc(cur, packed3d, zeros)
        cur, score = _layer_tc(
            cur, part, score, eps[i].reshape(1, 1),
            W1[i], b1[i].reshape(1, H), g1[i].reshape(1, H), be1[i].reshape(1, H),
            W2[i], b2[i].reshape(1, H), ga[i].reshape(1, H), ba[i].reshape(1, H),
            gl[i].reshape(1, H), bl[i].reshape(1, H), predW[i + 1])
    return score
